# baseline (speedup 1.0000x reference)
import functools

import jax
import jax.numpy as jnp
from jax import lax
from jax.experimental import pallas as pl
from jax.experimental.pallas import tpu as pltpu
from jax.experimental.pallas import tpu_sc as plsc

BATCH = 16
TOTAL_TOKENS = 32768
D_IN = 512
D_OUT = 64

NC = 2
NS = 16
NW = NC * NS
NSPLIT = 2
T_SPLIT = TOTAL_TOKENS // NSPLIT
ROWS_PER_W = T_SPLIT // NW
CHUNK = 512
NCH = ROWS_PER_W // CHUNK

MM_BLK = 2048


def _tc_matmul(flat_ref, w_ref, y_ref):
    y_ref[...] = lax.dot_general(
        flat_ref[...], w_ref[...], (((1,), (1,)), ((), ())),
        preferred_element_type=jnp.float32,
    )


_sc_mesh = plsc.VectorSubcoreMesh(
    core_axis_name="c", subcore_axis_name="s", num_cores=NC, num_subcores=NS
)


@functools.partial(
    pl.kernel,
    out_type=(
        jax.ShapeDtypeStruct((NW, BATCH, D_OUT), jnp.float32),
        jax.ShapeDtypeStruct((NW, BATCH, 16), jnp.float32),
    ),
    mesh=_sc_mesh,
    scratch_types=[
        pltpu.VMEM((ROWS_PER_W,), jnp.int32),
        pltpu.VMEM((CHUNK, D_OUT), jnp.float32),
        pltpu.VMEM((BATCH, D_OUT), jnp.float32),
        pltpu.VMEM((BATCH, 16), jnp.float32),
    ],
)
def _sc_segment_sums(y_hbm, seg_hbm, out_sum, out_cnt, idx_v, buf, acc, accc):
    cid = lax.axis_index("c")
    sid = lax.axis_index("s")
    wid = sid * NC + cid

    pltpu.sync_copy(seg_hbm.at[pl.ds(wid * ROWS_PER_W, ROWS_PER_W)], idx_v)

    zero = jnp.zeros((16,), jnp.float32)
    for i in range(BATCH):
        for j in range(D_OUT // 16):
            acc[i, pl.ds(j * 16, 16)] = zero
        accc[i, :] = zero

    def _hist(g, _):
        ids = idx_v[pl.ds(g * 16, 16)]
        for s in range(BATCH):
            sel = jnp.where(ids == s, 1.0, 0.0)
            plsc.addupdate(accc.at[s, :], sel)
        return 0

    lax.fori_loop(0, ROWS_PER_W // 16, _hist, 0)

    base = wid * ROWS_PER_W

    def _chunk(j, _):
        pltpu.sync_copy(y_hbm.at[pl.ds(base + j * CHUNK, CHUNK)], buf)

        def _grp(g, _):
            seg16 = idx_v[pl.ds((j * (CHUNK // 16) + g) * 16, 16)]
            for t in range(16):
                s = seg16[t]
                r = g * 16 + t
                for jb in range(D_OUT // 16):
                    plsc.addupdate(acc.at[s, pl.ds(jb * 16, 16)],
                                   buf[r, pl.ds(jb * 16, 16)])
            return 0

        lax.fori_loop(0, CHUNK // 16, _grp, 0)
        return 0

    lax.fori_loop(0, NCH, _chunk, 0)

    pltpu.sync_copy(acc, out_sum.at[wid])
    pltpu.sync_copy(accc, out_cnt.at[wid])


def _tc_finish(ps0_ref, ps1_ref, pc0_ref, pc1_ref, b_ref, o_ref):
    sums = jnp.sum(ps0_ref[...], axis=0) + jnp.sum(ps1_ref[...], axis=0)
    cnts = jnp.sum(pc0_ref[...], axis=0) + jnp.sum(pc1_ref[...], axis=0)
    cnt = jnp.sum(cnts, axis=1, keepdims=True)
    avg = sums / jnp.maximum(cnt, 1.0)
    o_ref[...] = avg + b_ref[...]


def _matmul_call(flat_h, W):
    return pl.pallas_call(
        _tc_matmul,
        grid=(T_SPLIT // MM_BLK,),
        in_specs=[
            pl.BlockSpec((MM_BLK, D_IN), lambda i: (i, 0)),
            pl.BlockSpec((D_OUT, D_IN), lambda i: (0, 0)),
        ],
        out_specs=pl.BlockSpec((MM_BLK, D_OUT), lambda i: (i, 0)),
        out_shape=jax.ShapeDtypeStruct((T_SPLIT, D_OUT), jnp.float32),
    )(flat_h, W)


def kernel(flat, segment_ids, W, b):
    seg = segment_ids.astype(jnp.int32)
    partials = []
    for h in range(NSPLIT):
        y_h = _matmul_call(
            lax.slice_in_dim(flat, h * T_SPLIT, (h + 1) * T_SPLIT, axis=0), W)
        partials.append(_sc_segment_sums(
            y_h, lax.slice_in_dim(seg, h * T_SPLIT, (h + 1) * T_SPLIT)))
    (ps0, pc0), (ps1, pc1) = partials
    out = pl.pallas_call(
        _tc_finish,
        out_shape=jax.ShapeDtypeStruct((BATCH, D_OUT), jnp.float32),
    )(ps0, ps1, pc0, pc1, b.reshape(1, D_OUT))
    return out


# speedup vs baseline: 1.6399x; 1.6399x over previous
import functools

import jax
import jax.numpy as jnp
from jax import lax
from jax.experimental import pallas as pl
from jax.experimental.pallas import tpu as pltpu
from jax.experimental.pallas import tpu_sc as plsc

BATCH = 16
TOTAL_TOKENS = 32768
D_IN = 512
D_OUT = 64

NC = 2
NS = 16
NW = NC * NS
NSPLIT = 2
T_SPLIT = TOTAL_TOKENS // NSPLIT
ROWS_PER_W = T_SPLIT // NW
CHUNK = 512
NCH = ROWS_PER_W // CHUNK

MM_BLK = 2048


def _tc_matmul(flat_ref, w_ref, y_ref):
    y_ref[...] = lax.dot_general(
        flat_ref[...], w_ref[...], (((1,), (1,)), ((), ())),
        preferred_element_type=jnp.float32,
    )


_sc_mesh = plsc.VectorSubcoreMesh(
    core_axis_name="c", subcore_axis_name="s", num_cores=NC, num_subcores=NS
)


def _make_sc_segment_sums(h):
    @functools.partial(
        pl.kernel,
        out_type=(
            jax.ShapeDtypeStruct((NW, BATCH, D_OUT), jnp.float32),
            jax.ShapeDtypeStruct((NW, BATCH, 16), jnp.float32),
        ),
        mesh=_sc_mesh,
        scratch_types=[
            pltpu.VMEM((ROWS_PER_W,), jnp.int32),
            pltpu.VMEM((CHUNK, D_OUT), jnp.float32),
            pltpu.VMEM((BATCH, D_OUT), jnp.float32),
            pltpu.VMEM((BATCH, 16), jnp.float32),
        ],
        name=f"sc_segment_sums_h{h}",
    )
    def _sc_segment_sums(y_hbm, seg_hbm, out_sum, out_cnt,
                         idx_v, buf, acc, accc):
        cid = lax.axis_index("c")
        sid = lax.axis_index("s")
        wid = sid * NC + cid

        pltpu.sync_copy(
            seg_hbm.at[pl.ds(h * T_SPLIT + wid * ROWS_PER_W, ROWS_PER_W)],
            idx_v)

        zero = jnp.zeros((16,), jnp.float32)
        for i in range(BATCH):
            for j in range(D_OUT // 16):
                acc[i, pl.ds(j * 16, 16)] = zero
            accc[i, :] = zero

        def _hist(g, _):
            ids = idx_v[pl.ds(g * 16, 16)]
            for s in range(BATCH):
                sel = jnp.where(ids == s, 1.0, 0.0)
                plsc.addupdate(accc.at[s, :], sel)
            return 0

        lax.fori_loop(0, ROWS_PER_W // 16, _hist, 0)

        base = wid * ROWS_PER_W

        def _chunk(j, _):
            pltpu.sync_copy(y_hbm.at[pl.ds(base + j * CHUNK, CHUNK)], buf)

            def _grp(g, _):
                seg16 = idx_v[pl.ds((j * (CHUNK // 16) + g) * 16, 16)]
                for t in range(16):
                    s = seg16[t]
                    r = g * 16 + t
                    for jb in range(D_OUT // 16):
                        plsc.addupdate(acc.at[s, pl.ds(jb * 16, 16)],
                                       buf[r, pl.ds(jb * 16, 16)])
                return 0

            lax.fori_loop(0, CHUNK // 16, _grp, 0)
            return 0

        lax.fori_loop(0, NCH, _chunk, 0)

        pltpu.sync_copy(acc, out_sum.at[wid])
        pltpu.sync_copy(accc, out_cnt.at[wid])

    return _sc_segment_sums


_sc_calls = [_make_sc_segment_sums(h) for h in range(NSPLIT)]


def _tc_finish(ps0_ref, ps1_ref, pc0_ref, pc1_ref, b_ref, o_ref):
    sums = jnp.sum(ps0_ref[...], axis=0) + jnp.sum(ps1_ref[...], axis=0)
    cnts = jnp.sum(pc0_ref[...], axis=0) + jnp.sum(pc1_ref[...], axis=0)
    cnt = jnp.sum(cnts, axis=1, keepdims=True)
    avg = sums / jnp.maximum(cnt, 1.0)
    o_ref[...] = avg + b_ref[...]


def _matmul_call(flat, W, h):
    nblk = T_SPLIT // MM_BLK
    return pl.pallas_call(
        _tc_matmul,
        grid=(nblk,),
        in_specs=[
            pl.BlockSpec((MM_BLK, D_IN), lambda i, h=h: (h * nblk + i, 0)),
            pl.BlockSpec((D_OUT, D_IN), lambda i: (0, 0)),
        ],
        out_specs=pl.BlockSpec((MM_BLK, D_OUT), lambda i: (i, 0)),
        out_shape=jax.ShapeDtypeStruct((T_SPLIT, D_OUT), jnp.float32),
    )(flat, W)


def kernel(flat, segment_ids, W, b):
    seg = segment_ids.astype(jnp.int32)
    partials = []
    for h in range(NSPLIT):
        y_h = _matmul_call(flat, W, h)
        partials.append(_sc_calls[h](y_h, seg))
    (ps0, pc0), (ps1, pc1) = partials
    out = pl.pallas_call(
        _tc_finish,
        out_shape=jax.ShapeDtypeStruct((BATCH, D_OUT), jnp.float32),
    )(ps0, ps1, pc0, pc1, b.reshape(1, D_OUT))
    return out
